# baseline (speedup 1.0000x reference)
import functools

import jax
import jax.numpy as jnp
from jax.experimental import pallas as pl
from jax.experimental.pallas import tpu as pltpu

NUM_CLAUSES = 8192
NUM_LITS = 4096
NUM_VARS = NUM_LITS // 2
D = 128
NUM_ROUNDS = 4

BC = 1024
BL = 512


def _dot(a, b):
    return jnp.dot(a, b, preferred_element_type=jnp.float32)


def _clause_body(g_ref, l_ref, c_ref, w1_ref, b1_ref, w2_ref, b2_ref,
                 sc_ref, o_ref):
    msgs = _dot(g_ref[...], l_ref[...]) * sc_ref[0]
    x = jnp.concatenate([c_ref[...], msgs], axis=1)
    h = jnp.maximum(_dot(x, w1_ref[...]) + b1_ref[...], 0.0)
    o_ref[...] = _dot(h, w2_ref[...]) + b2_ref[...]


def _literal_body(g_ref, c_ref, l_ref, lf_ref, w1_ref, b1_ref, w2_ref,
                  b2_ref, sc_ref, o_ref):
    msgs = jax.lax.dot_general(
        g_ref[...], c_ref[...],
        dimension_numbers=(((0,), (0,)), ((), ())),
        preferred_element_type=jnp.float32) * sc_ref[0]
    x = jnp.concatenate([l_ref[...], msgs, lf_ref[...]], axis=1)
    h = jnp.maximum(_dot(x, w1_ref[...]) + b1_ref[...], 0.0)
    o_ref[...] = _dot(h, w2_ref[...]) + b2_ref[...]


def _vote_body(l_ref, w1_ref, b1_ref, w2_ref, b2_ref, w3_ref,
               b3_ref, w4_ref, b4_ref, o_ref):
    v = jnp.concatenate([l_ref[:NUM_VARS, :], l_ref[NUM_VARS:, :]], axis=1)
    h = jnp.maximum(_dot(v, w1_ref[...]) + b1_ref[...], 0.0)
    h = jnp.maximum(_dot(h, w2_ref[...]) + b2_ref[...], 0.0)
    h = jnp.maximum(_dot(h, w3_ref[...]) + b3_ref[...], 0.0)
    o_ref[...] = _dot(h, w4_ref[...]) + b4_ref[...]


def _clause_phase(g, l, c, w1, b1, w2, b2, scale):
    nblk = NUM_CLAUSES // BC
    return pl.pallas_call(
        _clause_body,
        grid=(nblk,),
        in_specs=[
            pl.BlockSpec((BC, NUM_LITS), lambda i: (i, 0)),
            pl.BlockSpec((NUM_LITS, D), lambda i: (0, 0)),
            pl.BlockSpec((BC, D), lambda i: (i, 0)),
            pl.BlockSpec((2 * D, D), lambda i: (0, 0)),
            pl.BlockSpec((1, D), lambda i: (0, 0)),
            pl.BlockSpec((D, D), lambda i: (0, 0)),
            pl.BlockSpec((1, D), lambda i: (0, 0)),
            pl.BlockSpec(memory_space=pltpu.SMEM),
        ],
        out_specs=pl.BlockSpec((BC, D), lambda i: (i, 0)),
        out_shape=jax.ShapeDtypeStruct((NUM_CLAUSES, D), jnp.float32),
    )(g, l, c, w1, b1, w2, b2, scale)


def _literal_phase(g, c, l, w1, b1, w2, b2, scale):
    nblk = NUM_LITS // BL
    half = nblk // 2
    return pl.pallas_call(
        _literal_body,
        grid=(nblk,),
        in_specs=[
            pl.BlockSpec((NUM_CLAUSES, BL), lambda j: (0, j)),
            pl.BlockSpec((NUM_CLAUSES, D), lambda j: (0, 0)),
            pl.BlockSpec((BL, D), lambda j: (j, 0)),
            pl.BlockSpec((BL, D), lambda j: ((j + half) % nblk, 0)),
            pl.BlockSpec((3 * D, D), lambda j: (0, 0)),
            pl.BlockSpec((1, D), lambda j: (0, 0)),
            pl.BlockSpec((D, D), lambda j: (0, 0)),
            pl.BlockSpec((1, D), lambda j: (0, 0)),
            pl.BlockSpec(memory_space=pltpu.SMEM),
        ],
        out_specs=pl.BlockSpec((BL, D), lambda j: (j, 0)),
        out_shape=jax.ShapeDtypeStruct((NUM_LITS, D), jnp.float32),
    )(g, c, l, l, w1, b1, w2, b2, scale)


def _vote_phase(l, v_params):
    (w1, b1), (w2, b2), (w3, b3), (w4, b4) = v_params
    args = (l, w1, b1.reshape(1, D), w2, b2.reshape(1, D), w3,
            b3.reshape(1, D), w4, b4.reshape(1, 1))
    return pl.pallas_call(
        _vote_body,
        in_specs=[
            pl.BlockSpec(a.shape, functools.partial(lambda n: (0,) * n, a.ndim))
            for a in args
        ],
        out_specs=pl.BlockSpec((NUM_VARS, 1), lambda: (0, 0)),
        out_shape=jax.ShapeDtypeStruct((NUM_VARS, 1), jnp.float32),
    )(*args)


def kernel(G, c_params, l_params, v_params, c_init_scale, l_init_scale,
           cl_scale, lc_scale):
    L = jnp.full((NUM_LITS, D), 1.0, jnp.float32) * l_init_scale
    C = jnp.full((NUM_CLAUSES, D), 1.0, jnp.float32) * c_init_scale
    lc = lc_scale.reshape(1)
    cl = cl_scale.reshape(1)

    for r in range(NUM_ROUNDS):
        (w1c, b1c), (w2c, b2c) = c_params[r]
        C = _clause_phase(G, L, C, w1c, b1c.reshape(1, D),
                          w2c, b2c.reshape(1, D), lc)
        (w1l, b1l), (w2l, b2l) = l_params[r]
        L = _literal_phase(G, C, L, w1l, b1l.reshape(1, D),
                           w2l, b2l.reshape(1, D), cl)

    return _vote_phase(L, v_params).reshape(NUM_VARS)
